# Initial kernel scaffold; baseline (speedup 1.0000x reference)
import jax
import jax.numpy as jnp
from jax.experimental import pallas as pl


def kernel(t, node_embeddings, edge_index, edge_weights, W1, b1, W2, b2, hop_coef):
    raise NotImplementedError("write your pallas kernel here")


# speedup vs baseline: 8.3264x; 8.3264x over previous
import functools

import jax
import jax.numpy as jnp
from jax import lax
from jax.experimental import pallas as pl
from jax.experimental.pallas import tpu as pltpu
from jax.experimental.pallas import tpu_sc as plsc

N = 10000
NPAD = 10240
C = 128
NC = 2
NS = 16
NW = NC * NS
ECH = 80
RPS = NPAD // NS
ZB = 128


def _sc_partials(x, src, dst, w, lut, epw):
    nchunk = epw // ECH
    mesh = plsc.VectorSubcoreMesh(core_axis_name="c", subcore_axis_name="s")

    @functools.partial(
        pl.kernel,
        out_type=jax.ShapeDtypeStruct((2, NPAD, C), jnp.float32),
        mesh=mesh,
        scratch_types=[
            pltpu.VMEM((ECH,), jnp.int32),
            pltpu.VMEM((ECH,), jnp.int32),
            pltpu.VMEM((ECH,), jnp.int32),
            pltpu.VMEM((ECH,), jnp.float32),
            pltpu.VMEM((16,), jnp.float32),
            pltpu.VMEM((ECH, C), jnp.float32),
            pltpu.VMEM((ZB, C), jnp.float32),
            pltpu.VMEM_SHARED((NPAD, C), jnp.float32),
            pltpu.SemaphoreType.DMA,
        ],
        compiler_params=pltpu.CompilerParams(needs_layout_passes=False),
    )
    def sc_kernel(x_hbm, src_hbm, dst_hbm, w_hbm, lut_hbm, out_hbm,
                  src_v, dst_v, w_v, coef_v, lut_v, rows_v, zb_v, agg_sh, sem):
        cid = lax.axis_index("c")
        sid = lax.axis_index("s")
        wid = sid * NC + cid

        zeros16 = jnp.zeros((16,), jnp.float32)

        def zrow(i, carry):
            for j in range(C // 16):
                zb_v[i, pl.ds(j * 16, 16)] = zeros16
            return carry

        lax.fori_loop(0, ZB, zrow, 0)

        row0 = sid * RPS
        for k in range(RPS // ZB):
            pltpu.sync_copy(zb_v, agg_sh.at[pl.ds(row0 + k * ZB, ZB)])
        plsc.subcore_barrier()

        pltpu.sync_copy(lut_hbm, lut_v)

        ebase = wid * epw

        def chunk(c_i, carry):
            base = ebase + c_i * ECH
            pltpu.sync_copy(src_hbm.at[pl.ds(base, ECH)], src_v)
            pltpu.sync_copy(dst_hbm.at[pl.ds(base, ECH)], dst_v)
            pltpu.sync_copy(w_hbm.at[pl.ds(base, ECH)], w_v)
            pltpu.async_copy(x_hbm.at[src_v], rows_v, sem).wait()
            for g in range(ECH // 16):
                wv = w_v[pl.ds(g * 16, 16)]
                coef_v[pl.ds(g * 16, 16)] = plsc.load_gather(lut_v, [wv])

            def srow(r, rcarry):
                cv = plsc.load_gather(coef_v, [jnp.full((16,), r, jnp.int32)])
                for j in range(C // 16):
                    rows_v[r, pl.ds(j * 16, 16)] = (
                        rows_v[r, pl.ds(j * 16, 16)] * cv)
                return rcarry

            lax.fori_loop(0, ECH, srow, 0)
            pltpu.sync_copy(rows_v, agg_sh.at[dst_v], add=True)
            return carry

        lax.fori_loop(0, nchunk, chunk, 0)
        plsc.subcore_barrier()

        pltpu.sync_copy(agg_sh.at[pl.ds(row0, RPS)],
                        out_hbm.at[cid].at[pl.ds(row0, RPS)])

    return sc_kernel(x, src, dst, w, lut)


def _mlp_body(x_ref, p0_ref, p1_ref, w1_ref, b1_ref, w2_ref, b2_ref, o_ref):
    agg = x_ref[...] + p0_ref[0] + p1_ref[0]
    h = jnp.dot(agg, w1_ref[...], preferred_element_type=jnp.float32)
    h = jnp.maximum(h + b1_ref[...], 0.0)
    o = jnp.dot(h, w2_ref[...], preferred_element_type=jnp.float32)
    o_ref[...] = jnp.maximum(o + b2_ref[...], 0.0)


def _mlp(x, partials, W1, b1, W2, b2):
    BN = 1000
    grid = (N // BN,)
    return pl.pallas_call(
        _mlp_body,
        grid=grid,
        in_specs=[
            pl.BlockSpec((BN, C), lambda i: (i, 0)),
            pl.BlockSpec((1, BN, C), lambda i: (0, i, 0)),
            pl.BlockSpec((1, BN, C), lambda i: (1, i, 0)),
            pl.BlockSpec((C, C), lambda i: (0, 0)),
            pl.BlockSpec((1, C), lambda i: (0, 0)),
            pl.BlockSpec((C, C), lambda i: (0, 0)),
            pl.BlockSpec((1, C), lambda i: (0, 0)),
        ],
        out_specs=pl.BlockSpec((BN, C), lambda i: (i, 0)),
        out_shape=jax.ShapeDtypeStruct((N, C), jnp.float32),
    )(x, partials, partials, W1, b1.reshape(1, C), W2, b2.reshape(1, C))


def kernel(t, node_embeddings, edge_index, edge_weights, W1, b1, W2, b2,
           hop_coef):
    x = jnp.take(node_embeddings, t, axis=0)
    dst = edge_index[0]
    src = edge_index[1]
    lut = jnp.zeros((16,), jnp.float32).at[1:5].set(hop_coef)
    epw = edge_weights.shape[0] // NW
    partials = _sc_partials(x, src, dst, edge_weights, lut, epw)
    return _mlp(x, partials, W1, b1, W2, b2)


# speedup vs baseline: 18.7675x; 2.2540x over previous
import functools

import jax
import jax.numpy as jnp
from jax import lax
from jax.experimental import pallas as pl
from jax.experimental.pallas import tpu as pltpu
from jax.experimental.pallas import tpu_sc as plsc

N = 10000
NPAD = 10240
C = 128
NC = 2
NS = 16
NW = NC * NS
ECH = 80
RPS = NPAD // NS


def _sc_partials(x, packed, lut, epw):
    nchunk = epw // ECH
    npair = (nchunk - 1) // 2
    mesh = plsc.VectorSubcoreMesh(core_axis_name="c", subcore_axis_name="s")

    @functools.partial(
        pl.kernel,
        out_type=jax.ShapeDtypeStruct((2, NPAD, C), jnp.float32),
        mesh=mesh,
        scratch_types=[
            pltpu.VMEM((epw,), jnp.int32),
            pltpu.VMEM((16,), jnp.float32),
            pltpu.VMEM((ECH, C), jnp.float32),
            pltpu.VMEM((ECH, C), jnp.float32),
            pltpu.VMEM((ECH,), jnp.int32),
            pltpu.VMEM((ECH,), jnp.int32),
            pltpu.VMEM((ECH,), jnp.int32),
            pltpu.VMEM((ECH,), jnp.int32),
            pltpu.VMEM((ECH,), jnp.float32),
            pltpu.VMEM((ECH,), jnp.float32),
            pltpu.VMEM_SHARED((NPAD, C), jnp.float32),
            pltpu.SemaphoreType.DMA,
            pltpu.SemaphoreType.DMA,
        ],
        compiler_params=pltpu.CompilerParams(needs_layout_passes=False),
    )
    def sc_kernel(x_hbm, pk_hbm, lut_hbm, out_hbm,
                  pk_v, lut_v, rows_a, rows_b, src_a, src_b, dst_a, dst_b,
                  cf_a, cf_b, agg_sh, sem_a, sem_b):
        cid = lax.axis_index("c")
        sid = lax.axis_index("s")
        wid = sid * NC + cid

        pltpu.sync_copy(lut_hbm, lut_v)
        pltpu.sync_copy(pk_hbm.at[wid], pk_v)

        zeros16 = jnp.zeros((16,), jnp.float32)

        def zrow(i, carry):
            for j in range(C // 16):
                rows_a[i, pl.ds(j * 16, 16)] = zeros16
            return carry

        lax.fori_loop(0, ECH, zrow, 0)
        row0 = sid * RPS
        for k in range(RPS // ECH):
            pltpu.sync_copy(rows_a, agg_sh.at[pl.ds(row0 + k * ECH, ECH)])
        plsc.subcore_barrier()

        def decode(c_i, src_v, dst_v, cf_v):
            cbase = c_i * ECH
            for g in range(ECH // 16):
                pg = pk_v[pl.ds(cbase + g * 16, 16)]
                src_v[pl.ds(g * 16, 16)] = pg & 16383
                dst_v[pl.ds(g * 16, 16)] = (pg >> 14) & 16383
                cf_v[pl.ds(g * 16, 16)] = plsc.load_gather(
                    lut_v, [(pg >> 28) & 15])

        def gather(buf, src_v, sem):
            return pltpu.async_copy(x_hbm.at[src_v], buf, sem)

        def gwait(buf, src_v, sem):
            pltpu.make_async_copy(x_hbm.at[src_v], buf, sem).wait()

        def process(buf, dst_v, cf_v):
            def srow(r, rcarry):
                cv = plsc.load_gather(cf_v, [jnp.full((16,), 0, jnp.int32)
                                             + r])
                for j in range(C // 16):
                    buf[r, pl.ds(j * 16, 16)] = buf[r, pl.ds(j * 16, 16)] * cv
                return rcarry

            lax.fori_loop(0, ECH, srow, 0)
            pltpu.sync_copy(buf, agg_sh.at[dst_v], add=True)

        decode(0, src_a, dst_a, cf_a)
        gather(rows_a, src_a, sem_a)

        def pair(i, carry):
            c0 = 2 * i
            decode(c0 + 1, src_b, dst_b, cf_b)
            gwait(rows_a, src_a, sem_a)
            gather(rows_b, src_b, sem_b)
            process(rows_a, dst_a, cf_a)
            decode(c0 + 2, src_a, dst_a, cf_a)
            gwait(rows_b, src_b, sem_b)
            gather(rows_a, src_a, sem_a)
            process(rows_b, dst_b, cf_b)
            return carry

        lax.fori_loop(0, npair, pair, 0)
        gwait(rows_a, src_a, sem_a)
        process(rows_a, dst_a, cf_a)

        plsc.subcore_barrier()
        pltpu.sync_copy(agg_sh.at[pl.ds(row0, RPS)],
                        out_hbm.at[cid].at[pl.ds(row0, RPS)])

    return sc_kernel(x, packed, lut)


def _mlp_body(x_ref, p0_ref, p1_ref, w1_ref, b1_ref, w2_ref, b2_ref, o_ref):
    agg = x_ref[...] + p0_ref[0] + p1_ref[0]
    h = jnp.dot(agg, w1_ref[...], preferred_element_type=jnp.float32)
    h = jnp.maximum(h + b1_ref[...], 0.0)
    o = jnp.dot(h, w2_ref[...], preferred_element_type=jnp.float32)
    o_ref[...] = jnp.maximum(o + b2_ref[...], 0.0)


def _mlp(x, partials, W1, b1, W2, b2):
    BN = 1000
    grid = (N // BN,)
    return pl.pallas_call(
        _mlp_body,
        grid=grid,
        in_specs=[
            pl.BlockSpec((BN, C), lambda i: (i, 0)),
            pl.BlockSpec((1, BN, C), lambda i: (0, i, 0)),
            pl.BlockSpec((1, BN, C), lambda i: (1, i, 0)),
            pl.BlockSpec((C, C), lambda i: (0, 0)),
            pl.BlockSpec((1, C), lambda i: (0, 0)),
            pl.BlockSpec((C, C), lambda i: (0, 0)),
            pl.BlockSpec((1, C), lambda i: (0, 0)),
        ],
        out_specs=pl.BlockSpec((BN, C), lambda i: (i, 0)),
        out_shape=jax.ShapeDtypeStruct((N, C), jnp.float32),
    )(x, partials, partials, W1, b1.reshape(1, C), W2, b2.reshape(1, C))


def kernel(t, node_embeddings, edge_index, edge_weights, W1, b1, W2, b2,
           hop_coef):
    x = jnp.take(node_embeddings, t, axis=0)
    E = edge_weights.shape[0]
    epw = E // NW
    dst = edge_index[0]
    src = edge_index[1]
    packed = (src | (dst << 14) | (edge_weights << 28)).reshape(NW, epw)
    lut = jnp.zeros((16,), jnp.float32).at[1:5].set(hop_coef)
    partials = _sc_partials(x, packed, lut, epw)
    return _mlp(x, partials, W1, b1, W2, b2)


# speedup vs baseline: 23.5248x; 1.2535x over previous
import functools

import jax
import jax.numpy as jnp
from jax import lax
from jax.experimental import pallas as pl
from jax.experimental.pallas import tpu as pltpu
from jax.experimental.pallas import tpu_sc as plsc

N = 10000
NPAD = 10240
C = 128
NC = 2
NS = 16
NW = NC * NS
ECH = 80
RPS = NPAD // NS


def _sc_partials(x, packed, lut, epw):
    nchunk = epw // ECH
    ntrip = (nchunk - 2) // 3
    mesh = plsc.VectorSubcoreMesh(core_axis_name="c", subcore_axis_name="s")

    @functools.partial(
        pl.kernel,
        out_type=jax.ShapeDtypeStruct((2, NPAD, C), jnp.float32),
        mesh=mesh,
        scratch_types=[
            pltpu.VMEM((epw,), jnp.int32),
            pltpu.VMEM((16,), jnp.float32),
            [dict(rows=pltpu.VMEM((ECH, C), jnp.float32),
                  src=pltpu.VMEM((ECH,), jnp.int32),
                  dst=pltpu.VMEM((ECH,), jnp.int32),
                  cf=pltpu.VMEM((ECH,), jnp.float32),
                  gsem=pltpu.SemaphoreType.DMA,
                  ssem=pltpu.SemaphoreType.DMA) for _ in range(3)],
            pltpu.VMEM_SHARED((NPAD, C), jnp.float32),
        ],
        compiler_params=pltpu.CompilerParams(needs_layout_passes=False),
    )
    def sc_kernel(x_hbm, pk_hbm, lut_hbm, out_hbm, pk_v, lut_v, bufs, agg_sh):
        cid = lax.axis_index("c")
        sid = lax.axis_index("s")
        wid = sid * NC + cid
        b0, b1, b2 = bufs

        pltpu.sync_copy(lut_hbm, lut_v)
        pltpu.sync_copy(pk_hbm.at[wid], pk_v)

        zeros16 = jnp.zeros((16,), jnp.float32)

        def zrow(i, carry):
            for j in range(C // 16):
                b0["rows"][i, pl.ds(j * 16, 16)] = zeros16
            return carry

        lax.fori_loop(0, ECH, zrow, 0)
        row0 = sid * RPS
        for k in range(RPS // ECH):
            pltpu.sync_copy(b0["rows"],
                            agg_sh.at[pl.ds(row0 + k * ECH, ECH)])
        plsc.subcore_barrier()

        def decode(c_i, b):
            cbase = c_i * ECH
            for g in range(ECH // 16):
                pg = pk_v[pl.ds(cbase + g * 16, 16)]
                b["src"][pl.ds(g * 16, 16)] = pg & 16383
                b["dst"][pl.ds(g * 16, 16)] = (pg >> 14) & 16383
                b["cf"][pl.ds(g * 16, 16)] = plsc.load_gather(
                    lut_v, [(pg >> 28) & 15])

        def gather(b):
            pltpu.async_copy(x_hbm.at[b["src"]], b["rows"], b["gsem"])

        def gwait(b):
            pltpu.make_async_copy(x_hbm.at[b["src"]], b["rows"],
                                  b["gsem"]).wait()

        def scale(b):
            rows, cf = b["rows"], b["cf"]

            def srow(q, rcarry):
                r = 2 * q
                cv0 = plsc.load_gather(cf, [jnp.full((16,), 0, jnp.int32)
                                            + r])
                cv1 = plsc.load_gather(cf, [jnp.full((16,), 0, jnp.int32)
                                            + (r + 1)])
                for j in range(C // 16):
                    rows[r, pl.ds(j * 16, 16)] = (
                        rows[r, pl.ds(j * 16, 16)] * cv0)
                for j in range(C // 16):
                    rows[r + 1, pl.ds(j * 16, 16)] = (
                        rows[r + 1, pl.ds(j * 16, 16)] * cv1)
                return rcarry

            lax.fori_loop(0, ECH // 2, srow, 0)

        def scatter(b):
            pltpu.async_copy(b["rows"], agg_sh.at[b["dst"]], b["ssem"],
                             add=True)

        def swait(b):
            pltpu.make_async_copy(b["rows"], agg_sh.at[b["dst"]],
                                  b["ssem"]).wait()

        def step(c_i, bx, bz, first=False):
            gwait(bx)
            scale(bx)
            if not first:
                swait(bz)
            decode(jnp.minimum(c_i + 2, nchunk - 1), bz)
            gather(bz)
            scatter(bx)

        decode(0, b0)
        gather(b0)
        decode(1, b1)
        gather(b1)
        step(jnp.int32(0), b0, b2, first=True)

        def trip(t, carry):
            c_i = 3 * t + 1
            step(c_i, b1, b0)
            step(c_i + 1, b2, b1)
            step(c_i + 2, b0, b2)
            return carry

        lax.fori_loop(0, ntrip, trip, 0)

        gwait(b1)
        scale(b1)
        swait(b0)
        pltpu.sync_copy(b1["rows"], agg_sh.at[b1["dst"]], add=True)
        gwait(b2)

        plsc.subcore_barrier()
        pltpu.sync_copy(agg_sh.at[pl.ds(row0, RPS)],
                        out_hbm.at[cid].at[pl.ds(row0, RPS)])

    return sc_kernel(x, packed, lut)


def _mlp_body(x_ref, p0_ref, p1_ref, w1_ref, b1_ref, w2_ref, b2_ref, o_ref):
    agg = x_ref[...] + p0_ref[0] + p1_ref[0]
    h = jnp.dot(agg, w1_ref[...], preferred_element_type=jnp.float32)
    h = jnp.maximum(h + b1_ref[...], 0.0)
    o = jnp.dot(h, w2_ref[...], preferred_element_type=jnp.float32)
    o_ref[...] = jnp.maximum(o + b2_ref[...], 0.0)


def _mlp(x, partials, W1, b1, W2, b2):
    BN = 1000
    grid = (N // BN,)
    return pl.pallas_call(
        _mlp_body,
        grid=grid,
        in_specs=[
            pl.BlockSpec((BN, C), lambda i: (i, 0)),
            pl.BlockSpec((1, BN, C), lambda i: (0, i, 0)),
            pl.BlockSpec((1, BN, C), lambda i: (1, i, 0)),
            pl.BlockSpec((C, C), lambda i: (0, 0)),
            pl.BlockSpec((1, C), lambda i: (0, 0)),
            pl.BlockSpec((C, C), lambda i: (0, 0)),
            pl.BlockSpec((1, C), lambda i: (0, 0)),
        ],
        out_specs=pl.BlockSpec((BN, C), lambda i: (i, 0)),
        out_shape=jax.ShapeDtypeStruct((N, C), jnp.float32),
    )(x, partials, partials, W1, b1.reshape(1, C), W2, b2.reshape(1, C))


def kernel(t, node_embeddings, edge_index, edge_weights, W1, b1, W2, b2,
           hop_coef):
    x = jnp.take(node_embeddings, t, axis=0)
    E = edge_weights.shape[0]
    epw = E // NW
    dst = edge_index[0]
    src = edge_index[1]
    packed = (src | (dst << 14) | (edge_weights << 28)).reshape(NW, epw)
    lut = jnp.zeros((16,), jnp.float32).at[1:5].set(hop_coef)
    partials = _sc_partials(x, packed, lut, epw)
    return _mlp(x, partials, W1, b1, W2, b2)
